# baseline (speedup 1.0000x reference)
import functools

import jax
import jax.numpy as jnp
from jax import lax
from jax.experimental import pallas as pl
from jax.experimental.pallas import tpu as pltpu
from jax.experimental.pallas import tpu_sc as plsc

_ALPHA = 0.25

_N = 100000
_C = 80
_NP = 100096
_REAL = _N * _C
_PADN = (_NP - _N) * _C
_K = max(int(0.3 * _REAL), 1)

_BLK = 4352
_NB = -(-_NP // _BLK)

_BINS = 16384
_SHIFT = 17
_SQ = 128

_NC, _NS = 2, 16
_NW = _NC * _NS
_CCH = 256
_NCHT = _NP // _CCH
_PER_CH = -(-_NCHT // _NW)


def _loss_body(x_ref, t_ref, loss_ref, sum_ref):
    x = x_ref[...]
    t = t_ref[...]
    cls = lax.broadcasted_iota(jnp.int32, (_C, _BLK), 0)
    tt = jnp.where(cls == t, 1.0, 0.0)
    ax = jnp.abs(x)
    e = jnp.exp(-ax)
    opp = 1.0 + e
    l = jnp.log(opp)
    ps = jnp.where(x >= 0.0, 1.0, e) / opp
    pt = ps + tt - 2.0 * ps * tt
    aw = (1.0 - _ALPHA) + (2.0 * _ALPHA - 1.0) * tt
    bce = jnp.maximum(x, 0.0) - x * tt + l
    raw = aw * (pt * pt) * bce
    col = pl.program_id(0) * _BLK + lax.broadcasted_iota(jnp.int32,
                                                         (_C, _BLK), 1)
    loss = jnp.where(col < _N, raw, 0.0)
    loss_ref[...] = loss

    @pl.when(pl.program_id(0) == 0)
    def _():
        sum_ref[...] = jnp.zeros((1, 1), jnp.float32)

    sum_ref[...] += jnp.sum(loss, keepdims=True)


def _loss_call(xt, t):
    return pl.pallas_call(
        _loss_body,
        grid=(_NB,),
        in_specs=[
            pl.BlockSpec((_C, _BLK), lambda i: (0, i)),
            pl.BlockSpec((1, _BLK), lambda i: (0, i)),
        ],
        out_specs=[
            pl.BlockSpec((_C, _BLK), lambda i: (0, i)),
            pl.BlockSpec((1, 1), lambda i: (0, 0)),
        ],
        out_shape=[
            jax.ShapeDtypeStruct((_C, _NP), jnp.float32),
            jax.ShapeDtypeStruct((1, 1), jnp.float32),
        ],
    )(xt, t)


def _hist_body(loss_hbm, cnt_hbm, sum_hbm, buf, hcnt, hsum, hcnt2, hsum2,
               sem0, sem1):
    wid = lax.axis_index("s") * _NC + lax.axis_index("c")
    lo = jnp.minimum(wid * _PER_CH, _NCHT)
    n = jnp.minimum(lo + _PER_CH, _NCHT) - lo
    zeros = jnp.zeros((16,), jnp.float32)
    ones = jnp.ones((16,), jnp.float32)

    @plsc.parallel_loop(0, _SQ, unroll=8)
    def _(i):
        for sub in range(_SQ // 16):
            hcnt[i, pl.ds(sub * 16, 16)] = zeros
            hsum[i, pl.ds(sub * 16, 16)] = zeros
            hcnt2[i, pl.ds(sub * 16, 16)] = zeros
            hsum2[i, pl.ds(sub * 16, 16)] = zeros

    sems = [sem0, sem1]

    def start(ci, slot):
        pltpu.async_copy(loss_hbm.at[:, pl.ds((lo + ci) * _CCH, _CCH)],
                         buf.at[slot], sems[slot])

    def wait(slot):
        pltpu.make_async_copy(loss_hbm.at[:, pl.ds(0, _CCH)], buf.at[slot],
                              sems[slot]).wait()

    def process(slot):
        @plsc.parallel_loop(0, _C, unroll=4)
        def _(r):
            for sub in range(_CCH // 16):
                v = buf[slot, r, pl.ds(sub * 16, 16)]
                idx = lax.shift_right_logical(plsc.bitcast(v, jnp.int32),
                                              _SHIFT)
                hi = lax.shift_right_logical(idx, 7)
                lo_i = lax.bitwise_and(idx, 127)
                hc = hcnt if sub % 2 == 0 else hcnt2
                hs = hsum if sub % 2 == 0 else hsum2
                plsc.addupdate_scatter(hc, [hi, lo_i], ones)
                plsc.addupdate_scatter(hs, [hi, lo_i], v)

    @pl.when(n > 0)
    def _():
        start(0, 0)

    @pl.when(n > 1)
    def _():
        start(1, 1)

    def outer(g, c):
        for b in range(2):
            ci = g * 2 + b

            @pl.when(ci < n)
            def _():
                wait(b)
                process(b)

                @pl.when(ci + 2 < n)
                def _():
                    start(ci + 2, b)
        return c

    lax.fori_loop(0, (_PER_CH + 1) // 2, outer, 0)

    @plsc.parallel_loop(0, _SQ, unroll=4)
    def _(i):
        for sub in range(_SQ // 16):
            sl = pl.ds(sub * 16, 16)
            hcnt[i, sl] = hcnt[i, sl] + hcnt2[i, sl]
            hsum[i, sl] = hsum[i, sl] + hsum2[i, sl]

    pltpu.sync_copy(hcnt, cnt_hbm.at[wid])
    pltpu.sync_copy(hsum, sum_hbm.at[wid])


@functools.cache
def _make_hist_call():
    return functools.partial(
        pl.kernel,
        mesh=plsc.VectorSubcoreMesh(core_axis_name="c", subcore_axis_name="s"),
        out_type=[
            jax.ShapeDtypeStruct((_NW, _SQ, _SQ), jnp.float32),
            jax.ShapeDtypeStruct((_NW, _SQ, _SQ), jnp.float32),
        ],
        scratch_types=[
            pltpu.VMEM((2, _C, _CCH), jnp.float32),
            pltpu.VMEM((_SQ, _SQ), jnp.float32),
            pltpu.VMEM((_SQ, _SQ), jnp.float32),
            pltpu.VMEM((_SQ, _SQ), jnp.float32),
            pltpu.VMEM((_SQ, _SQ), jnp.float32),
            pltpu.SemaphoreType.DMA,
            pltpu.SemaphoreType.DMA,
        ],
        compiler_params=pltpu.CompilerParams(needs_layout_passes=False),
    )(_hist_body)


def _select_body(cnt_ref, sum_ref, tot_ref, out_ref):
    h = jnp.zeros((_SQ, _SQ), jnp.float32)
    s = jnp.zeros((_SQ, _SQ), jnp.float32)
    for w in range(_NW):
        h = h + cnt_ref[w]
        s = s + sum_ref[w]
    rows = lax.broadcasted_iota(jnp.int32, (_SQ, _SQ), 0)
    cols = lax.broadcasted_iota(jnp.int32, (_SQ, _SQ), 1)
    binid = rows * _SQ + cols
    h = h - jnp.where(binid == 0, float(_PADN), 0.0)

    hi = jax.lax.Precision.HIGHEST
    inc = jnp.where(rows <= cols, 1.0, 0.0)
    strict = jnp.where(cols < rows, 1.0, 0.0)

    def csum(m):
        prefix = jax.lax.dot(m, inc, precision=hi)
        row_tot = jnp.sum(m, axis=1, keepdims=True)
        prev = jax.lax.dot(strict, row_tot, precision=hi)
        return prefix + prev

    csum_h = csum(h)
    csum_s = csum(s)
    cnt_ge = float(_REAL) - csum_h + h
    bstar = jnp.sum((cnt_ge >= float(_K)).astype(jnp.int32)) - 1
    sel = binid == bstar
    hb = jnp.sum(jnp.where(sel, h, 0.0))
    sb = jnp.sum(jnp.where(sel, s, 0.0))
    csum_hb = jnp.sum(jnp.where(sel, csum_h, 0.0))
    csum_sb = jnp.sum(jnp.where(sel, csum_s, 0.0))
    cnt_gt = float(_REAL) - csum_hb
    sum_gt = jnp.sum(s) - csum_sb
    r = float(_K) - cnt_gt
    vb = sb / jnp.maximum(hb, 1.0)
    topk_sum = sum_gt + r * vb
    out_ref[...] = tot_ref[...] / float(_REAL) + jnp.full((1, 1), topk_sum / float(_K))


def _select_call(cnt, sm, tot):
    return pl.pallas_call(
        _select_body,
        out_shape=jax.ShapeDtypeStruct((1, 1), jnp.float32),
    )(cnt, sm, tot)


def kernel(input, target):
    xt = input.T
    t2 = target.reshape(1, _N)
    loss, tot = _loss_call(xt, t2)
    cnt, sm = _make_hist_call()(loss)
    res = _select_call(cnt, sm, tot)
    return res[0, 0]


# speedup vs baseline: 1.1179x; 1.1179x over previous
import functools

import jax
import jax.numpy as jnp
from jax import lax
from jax.experimental import pallas as pl
from jax.experimental.pallas import tpu as pltpu
from jax.experimental.pallas import tpu_sc as plsc

_ALPHA = 0.25

_N = 100000
_C = 80
_NP = 100096
_REAL = _N * _C
_PADN = (_NP - _N) * _C
_K = max(int(0.3 * _REAL), 1)

_NH = 50048
_BLK = 2944
_NBH = _NH // _BLK

_BINS = 16384
_SHIFT = 17
_SQ = 128

_NC, _NS = 2, 16
_NW = _NC * _NS
_CCH = 128
_NCHT = _NH // _CCH
_PER_CH = -(-_NCHT // _NW)


def _loss_body(base, x_ref, t_ref, loss_ref, sum_ref):
    x = x_ref[...]
    t = t_ref[...]
    cls = lax.broadcasted_iota(jnp.int32, (_C, _BLK), 0)
    tt = jnp.where(cls == t, 1.0, 0.0)
    ax = jnp.abs(x)
    e = jnp.exp(-ax)
    opp = 1.0 + e
    l = jnp.log(opp)
    ps = jnp.where(x >= 0.0, 1.0, e) / opp
    pt = ps + tt - 2.0 * ps * tt
    aw = (1.0 - _ALPHA) + (2.0 * _ALPHA - 1.0) * tt
    bce = jnp.maximum(x, 0.0) - x * tt + l
    raw = aw * (pt * pt) * bce
    col = base + pl.program_id(0) * _BLK + lax.broadcasted_iota(
        jnp.int32, (_C, _BLK), 1)
    loss = jnp.where(col < _N, raw, 0.0)
    loss_ref[...] = loss

    @pl.when(pl.program_id(0) == 0)
    def _():
        sum_ref[...] = jnp.zeros((1, 1), jnp.float32)

    sum_ref[...] += jnp.sum(loss, keepdims=True)


def _loss_call(xt, t, half):
    off = half * _NBH
    return pl.pallas_call(
        functools.partial(_loss_body, half * _NH),
        grid=(_NBH,),
        in_specs=[
            pl.BlockSpec((_C, _BLK), lambda i: (0, i + off)),
            pl.BlockSpec((1, _BLK), lambda i: (0, i + off)),
        ],
        out_specs=[
            pl.BlockSpec((_C, _BLK), lambda i: (0, i)),
            pl.BlockSpec((1, 1), lambda i: (0, 0)),
        ],
        out_shape=[
            jax.ShapeDtypeStruct((_C, _NH), jnp.float32),
            jax.ShapeDtypeStruct((1, 1), jnp.float32),
        ],
    )(xt, t)


def _hist_body(loss_hbm, cnt_hbm, sum_hbm, buf, hcnt, hsum, sem0, sem1):
    wid = lax.axis_index("s") * _NC + lax.axis_index("c")
    lo = jnp.minimum(wid * _PER_CH, _NCHT)
    n = jnp.minimum(lo + _PER_CH, _NCHT) - lo
    zeros = jnp.zeros((16,), jnp.float32)
    ones = jnp.ones((16,), jnp.float32)

    @plsc.parallel_loop(0, _SQ, unroll=8)
    def _(i):
        for sub in range(_SQ // 16):
            hcnt[i, pl.ds(sub * 16, 16)] = zeros
            hsum[i, pl.ds(sub * 16, 16)] = zeros

    sems = [sem0, sem1]

    def start(ci, slot):
        pltpu.async_copy(loss_hbm.at[:, pl.ds((lo + ci) * _CCH, _CCH)],
                         buf.at[slot], sems[slot])

    def wait(slot):
        pltpu.make_async_copy(loss_hbm.at[:, pl.ds(0, _CCH)], buf.at[slot],
                              sems[slot]).wait()

    def process(slot):
        @plsc.parallel_loop(0, _C, unroll=4)
        def _(r):
            for sub in range(_CCH // 16):
                v = buf[slot, r, pl.ds(sub * 16, 16)]
                idx = lax.shift_right_logical(plsc.bitcast(v, jnp.int32),
                                              _SHIFT)
                hi = lax.shift_right_logical(idx, 7)
                lo_i = lax.bitwise_and(idx, 127)
                plsc.addupdate_scatter(hcnt, [hi, lo_i], ones)
                plsc.addupdate_scatter(hsum, [hi, lo_i], v)

    @pl.when(n > 0)
    def _():
        start(0, 0)

    @pl.when(n > 1)
    def _():
        start(1, 1)

    def outer(g, c):
        for b in range(2):
            ci = g * 2 + b

            @pl.when(ci < n)
            def _():
                wait(b)
                process(b)

                @pl.when(ci + 2 < n)
                def _():
                    start(ci + 2, b)
        return c

    lax.fori_loop(0, (_PER_CH + 1) // 2, outer, 0)
    pltpu.sync_copy(hcnt, cnt_hbm.at[wid])
    pltpu.sync_copy(hsum, sum_hbm.at[wid])


@functools.cache
def _make_hist_call():
    return functools.partial(
        pl.kernel,
        mesh=plsc.VectorSubcoreMesh(core_axis_name="c", subcore_axis_name="s"),
        out_type=[
            jax.ShapeDtypeStruct((_NW, _SQ, _SQ), jnp.float32),
            jax.ShapeDtypeStruct((_NW, _SQ, _SQ), jnp.float32),
        ],
        scratch_types=[
            pltpu.VMEM((2, _C, _CCH), jnp.float32),
            pltpu.VMEM((_SQ, _SQ), jnp.float32),
            pltpu.VMEM((_SQ, _SQ), jnp.float32),
            pltpu.SemaphoreType.DMA,
            pltpu.SemaphoreType.DMA,
        ],
        compiler_params=pltpu.CompilerParams(needs_layout_passes=False),
    )(_hist_body)


def _select_body(cnt_ref, sum_ref, cnt_ref2, sum_ref2, tot_ref, out_ref):
    h = jnp.zeros((_SQ, _SQ), jnp.float32)
    s = jnp.zeros((_SQ, _SQ), jnp.float32)
    for w in range(_NW):
        h = h + cnt_ref[w] + cnt_ref2[w]
        s = s + sum_ref[w] + sum_ref2[w]
    rows = lax.broadcasted_iota(jnp.int32, (_SQ, _SQ), 0)
    cols = lax.broadcasted_iota(jnp.int32, (_SQ, _SQ), 1)
    binid = rows * _SQ + cols
    h = h - jnp.where(binid == 0, float(_PADN), 0.0)

    hi = jax.lax.Precision.HIGHEST
    inc = jnp.where(rows <= cols, 1.0, 0.0)
    strict = jnp.where(cols < rows, 1.0, 0.0)

    def csum(m):
        prefix = jax.lax.dot(m, inc, precision=hi)
        row_tot = jnp.sum(m, axis=1, keepdims=True)
        prev = jax.lax.dot(strict, row_tot, precision=hi)
        return prefix + prev

    csum_h = csum(h)
    csum_s = csum(s)
    cnt_ge = float(_REAL) - csum_h + h
    bstar = jnp.sum((cnt_ge >= float(_K)).astype(jnp.int32)) - 1
    sel = binid == bstar
    hb = jnp.sum(jnp.where(sel, h, 0.0))
    sb = jnp.sum(jnp.where(sel, s, 0.0))
    csum_hb = jnp.sum(jnp.where(sel, csum_h, 0.0))
    csum_sb = jnp.sum(jnp.where(sel, csum_s, 0.0))
    cnt_gt = float(_REAL) - csum_hb
    sum_gt = jnp.sum(s) - csum_sb
    r = float(_K) - cnt_gt
    vb = sb / jnp.maximum(hb, 1.0)
    topk_sum = sum_gt + r * vb
    out_ref[...] = tot_ref[...] / float(_REAL) + jnp.full((1, 1), topk_sum / float(_K))


def _select_call(cnt, sm, cnt2, sm2, tot):
    return pl.pallas_call(
        _select_body,
        out_shape=jax.ShapeDtypeStruct((1, 1), jnp.float32),
    )(cnt, sm, cnt2, sm2, tot)


def kernel(input, target):
    xt = input.T
    t2 = target.reshape(1, _N)
    hist = _make_hist_call()
    loss1, tot1 = _loss_call(xt, t2, 0)
    cnt1, sm1 = hist(loss1)
    loss2, tot2 = _loss_call(xt, t2, 1)
    cnt2, sm2 = hist(loss2)
    res = _select_call(cnt1, sm1, cnt2, sm2, tot1 + tot2)
    return res[0, 0]


# speedup vs baseline: 1.4072x; 1.2588x over previous
import functools

import jax
import jax.numpy as jnp
from jax import lax
from jax.experimental import pallas as pl
from jax.experimental.pallas import tpu as pltpu
from jax.experimental.pallas import tpu_sc as plsc

_ALPHA = 0.25

_N = 100000
_C = 80
_NP = 100096
_REAL = _N * _C
_PADN = (_NP - _N) * _C
_K = max(int(0.3 * _REAL), 1)

_NH = 50048
_BLK = 2944
_NBH = _NH // _BLK

_BINS = 16384
_SHIFT = 17
_SQ = 128

_NC, _NS = 2, 16
_NW = _NC * _NS
_CCH = 128
_NCHT = _NH // _CCH
_PER_CH = -(-_NCHT // _NW)


def _loss_body(base, x_ref, t_ref, loss_ref, sum_ref):
    x = x_ref[...]
    t = t_ref[...]
    cls = lax.broadcasted_iota(jnp.int32, (_C, _BLK), 0)
    tt = jnp.where(cls == t, 1.0, 0.0)
    ax = jnp.abs(x)
    e = jnp.exp(-ax)
    opp = 1.0 + e
    l = jnp.log(opp)
    ps = jnp.where(x >= 0.0, 1.0, e) / opp
    pt = ps + tt - 2.0 * ps * tt
    aw = (1.0 - _ALPHA) + (2.0 * _ALPHA - 1.0) * tt
    bce = jnp.maximum(x, 0.0) - x * tt + l
    raw = aw * (pt * pt) * bce
    col = base + pl.program_id(0) * _BLK + lax.broadcasted_iota(
        jnp.int32, (_C, _BLK), 1)
    loss = jnp.where(col < _N, raw, 0.0)
    loss_ref[...] = loss

    @pl.when(pl.program_id(0) == 0)
    def _():
        sum_ref[...] = jnp.zeros((1, 1), jnp.float32)

    sum_ref[...] += jnp.sum(loss, keepdims=True)


def _loss_call(xt, t, half):
    off = half * _NBH
    return pl.pallas_call(
        functools.partial(_loss_body, half * _NH),
        grid=(_NBH,),
        in_specs=[
            pl.BlockSpec((_C, _BLK), lambda i: (0, i + off)),
            pl.BlockSpec((1, _BLK), lambda i: (0, i + off)),
        ],
        out_specs=[
            pl.BlockSpec((_C, _BLK), lambda i: (0, i)),
            pl.BlockSpec((1, 1), lambda i: (0, 0)),
        ],
        out_shape=[
            jax.ShapeDtypeStruct((_C, _NH), jnp.float32),
            jax.ShapeDtypeStruct((1, 1), jnp.float32),
        ],
    )(xt, t)


def _hist_body(loss_hbm, cnt_hbm, buf, hcnt, sem0, sem1):
    wid = lax.axis_index("s") * _NC + lax.axis_index("c")
    lo = jnp.minimum(wid * _PER_CH, _NCHT)
    n = jnp.minimum(lo + _PER_CH, _NCHT) - lo
    zeros = jnp.zeros((16,), jnp.float32)
    ones = jnp.ones((16,), jnp.float32)

    @plsc.parallel_loop(0, _SQ, unroll=8)
    def _(i):
        for sub in range(_SQ // 16):
            hcnt[i, pl.ds(sub * 16, 16)] = zeros

    sems = [sem0, sem1]

    def start(ci, slot):
        pltpu.async_copy(loss_hbm.at[:, pl.ds((lo + ci) * _CCH, _CCH)],
                         buf.at[slot], sems[slot])

    def wait(slot):
        pltpu.make_async_copy(loss_hbm.at[:, pl.ds(0, _CCH)], buf.at[slot],
                              sems[slot]).wait()

    def process(slot):
        @plsc.parallel_loop(0, _C, unroll=4)
        def _(r):
            for sub in range(_CCH // 16):
                v = buf[slot, r, pl.ds(sub * 16, 16)]
                idx = lax.shift_right_logical(plsc.bitcast(v, jnp.int32),
                                              _SHIFT)
                hi = lax.shift_right_logical(idx, 7)
                lo_i = lax.bitwise_and(idx, 127)
                plsc.addupdate_scatter(hcnt, [hi, lo_i], ones)

    @pl.when(n > 0)
    def _():
        start(0, 0)

    @pl.when(n > 1)
    def _():
        start(1, 1)

    def outer(g, c):
        for b in range(2):
            ci = g * 2 + b

            @pl.when(ci < n)
            def _():
                wait(b)
                process(b)

                @pl.when(ci + 2 < n)
                def _():
                    start(ci + 2, b)
        return c

    lax.fori_loop(0, (_PER_CH + 1) // 2, outer, 0)
    pltpu.sync_copy(hcnt, cnt_hbm.at[wid])


@functools.cache
def _make_hist_call():
    return functools.partial(
        pl.kernel,
        mesh=plsc.VectorSubcoreMesh(core_axis_name="c", subcore_axis_name="s"),
        out_type=jax.ShapeDtypeStruct((_NW, _SQ, _SQ), jnp.float32),
        scratch_types=[
            pltpu.VMEM((2, _C, _CCH), jnp.float32),
            pltpu.VMEM((_SQ, _SQ), jnp.float32),
            pltpu.SemaphoreType.DMA,
            pltpu.SemaphoreType.DMA,
        ],
        compiler_params=pltpu.CompilerParams(needs_layout_passes=False),
    )(_hist_body)


def _select_body(cnt_ref, cnt_ref2, tot_ref, out_ref):
    h = jnp.zeros((_SQ, _SQ), jnp.float32)
    for w in range(_NW):
        h = h + cnt_ref[w] + cnt_ref2[w]
    rows = lax.broadcasted_iota(jnp.int32, (_SQ, _SQ), 0)
    cols = lax.broadcasted_iota(jnp.int32, (_SQ, _SQ), 1)
    binid = rows * _SQ + cols
    h = h - jnp.where(binid == 0, float(_PADN), 0.0)
    rep = lax.bitcast_convert_type(binid * (1 << _SHIFT) + (1 << (_SHIFT - 1)),
                                   jnp.float32)
    s = h * rep

    hi = jax.lax.Precision.HIGHEST
    inc = jnp.where(rows <= cols, 1.0, 0.0)
    strict = jnp.where(cols < rows, 1.0, 0.0)

    def csum(m):
        prefix = jax.lax.dot(m, inc, precision=hi)
        row_tot = jnp.sum(m, axis=1, keepdims=True)
        prev = jax.lax.dot(strict, row_tot, precision=hi)
        return prefix + prev

    csum_h = csum(h)
    csum_s = csum(s)
    cnt_ge = float(_REAL) - csum_h + h
    bstar = jnp.sum((cnt_ge >= float(_K)).astype(jnp.int32)) - 1
    sel = binid == bstar
    hb = jnp.sum(jnp.where(sel, h, 0.0))
    sb = jnp.sum(jnp.where(sel, s, 0.0))
    csum_hb = jnp.sum(jnp.where(sel, csum_h, 0.0))
    csum_sb = jnp.sum(jnp.where(sel, csum_s, 0.0))
    cnt_gt = float(_REAL) - csum_hb
    sum_gt = jnp.sum(s) - csum_sb
    r = float(_K) - cnt_gt
    vb = sb / jnp.maximum(hb, 1.0)
    topk_sum = sum_gt + r * vb
    out_ref[...] = tot_ref[...] / float(_REAL) + jnp.full((1, 1), topk_sum / float(_K))


def _select_call(cnt, cnt2, tot):
    return pl.pallas_call(
        _select_body,
        out_shape=jax.ShapeDtypeStruct((1, 1), jnp.float32),
    )(cnt, cnt2, tot)


def kernel(input, target):
    xt = input.T
    t2 = target.reshape(1, _N)
    hist = _make_hist_call()
    loss1, tot1 = _loss_call(xt, t2, 0)
    cnt1 = hist(loss1)
    loss2, tot2 = _loss_call(xt, t2, 1)
    cnt2 = hist(loss2)
    res = _select_call(cnt1, cnt2, tot1 + tot2)
    return res[0, 0]
